# Initial kernel scaffold; baseline (speedup 1.0000x reference)
import jax
import jax.numpy as jnp
from jax.experimental import pallas as pl


def kernel(x, gate_values, W, b):
    raise NotImplementedError("write your pallas kernel here")


# speedup vs baseline: 2.7397x; 2.7397x over previous
import jax
import jax.numpy as jnp
from jax.experimental import pallas as pl
from jax.experimental.pallas import tpu as pltpu

NUM_EXPERTS = 4
CH = 96
HW = 224
BH = 16
WPAD = 232
HPAD = 240


def _conv_kernel(gate_ref, w_ref, b_ref, xa_ref, xb_ref, out_ref):
    g = gate_ref[0]

    x = jnp.concatenate([xa_ref[0], xb_ref[0]], axis=0)

    accs = []
    for dw in range(3):
        acc = jnp.zeros((BH * WPAD, CH), dtype=jnp.float32)
        for dh in range(3):
            wm = jnp.zeros((CH, CH), dtype=jnp.float32)
            for e in range(NUM_EXPERTS):
                ge = g[0:1, e:e + 1]
                wm = wm + ge * w_ref[e, dh, dw]
            xs = x[dh:dh + BH].reshape(BH * WPAD, CH)
            acc = acc + jnp.dot(xs, wm, preferred_element_type=jnp.float32)
        accs.append(acc.reshape(BH, WPAD, CH))

    bm = jnp.dot(g, b_ref[...], preferred_element_type=jnp.float32)
    out = (accs[0][:, 0:HW] + accs[1][:, 1:HW + 1] + accs[2][:, 2:HW + 2]
           + bm[None, :, :])
    out_ref[0] = jnp.maximum(out, 0.0)


def kernel(x, gate_values, W, b):
    B = x.shape[0]
    xt = jnp.transpose(x, (0, 2, 3, 1))
    xp = jnp.pad(xt, ((0, 0), (1, HPAD - HW - 1), (1, WPAD - HW - 1), (0, 0)))
    wt = jnp.transpose(W, (0, 3, 4, 2, 1))
    gv = gate_values.reshape(B, 1, NUM_EXPERTS)

    n_h = HW // BH
    out = pl.pallas_call(
        _conv_kernel,
        grid=(B, n_h),
        in_specs=[
            pl.BlockSpec((1, 1, NUM_EXPERTS), lambda bb, h: (bb, 0, 0)),
            pl.BlockSpec((NUM_EXPERTS, 3, 3, CH, CH), lambda bb, h: (0, 0, 0, 0, 0)),
            pl.BlockSpec((NUM_EXPERTS, CH), lambda bb, h: (0, 0)),
            pl.BlockSpec((1, BH, WPAD, CH), lambda bb, h: (bb, h, 0, 0)),
            pl.BlockSpec((1, BH, WPAD, CH), lambda bb, h: (bb, h + 1, 0, 0)),
        ],
        out_specs=pl.BlockSpec((1, BH, HW, CH), lambda bb, h: (bb, h, 0, 0)),
        out_shape=jax.ShapeDtypeStruct((B, HW, HW, CH), jnp.float32),
        compiler_params=pltpu.CompilerParams(
            dimension_semantics=("parallel", "arbitrary"),
        ),
    )(gv, wt, b, xp, xp)
    return jnp.transpose(out, (0, 3, 1, 2))
